# baseline (speedup 1.0000x reference)
import jax
import jax.numpy as jnp
from jax.experimental import pallas as pl
from jax.experimental.pallas import tpu as pltpu

B, S, D = 8, 512, 768
C = 5
H1 = 768
H2 = 384
CP = 128

_C1 = 0.7978845608028654
_C2 = _C1 * 0.044715


def _gelu(x):
    t = jnp.tanh(x * (_C1 + _C2 * (x * x)))
    return x * (0.5 * t + 0.5)


PB = 1
G = B // PB
CHUNK = 128


def _softmax_rows(logits):
    m = jnp.max(logits, axis=-1, keepdims=True)
    e = jnp.exp(logits - m)
    return e / jnp.sum(e, axis=-1, keepdims=True)


def _mlp_body(nseg_ref, q_ref, seg_ref, tis_ref, w1q_ref, w1s_ref, b1_ref,
              w2_ref, b2_ref, w3_ref, b3_ref, probs_ref, tinst_ref,
              qh_s, pad_s):
    b = pl.program_id(0)

    @pl.when(b == 0)
    def _init():
        qh = jnp.dot(q_ref[...], w1q_ref[...],
                     preferred_element_type=jnp.float32)
        qh_s[...] = qh
        h1p = _gelu(qh + b1_ref[...])
        h2p = _gelu(jnp.dot(h1p, w2_ref[...],
                            preferred_element_type=jnp.float32) + b2_ref[...])
        lp = jnp.dot(h2p, w3_ref[...],
                     preferred_element_type=jnp.float32) + b3_ref[...]
        pad_s[...] = _softmax_rows(lp)

    n = nseg_ref[b]
    qh_b = qh_s[pl.ds(b, 1)]

    probs_ref[0] = jnp.broadcast_to(pad_s[pl.ds(b, 1), :C], (S, C))

    rowc = jax.lax.broadcasted_iota(jnp.int32, (CHUNK, 1), 0)

    def _chunk(k, carry):
        base = k * CHUNK
        xk = jnp.where(rowc + base < n, seg_ref[0, pl.ds(base, CHUNK), :],
                       0.0)
        h1 = _gelu(jnp.dot(xk, w1s_ref[...],
                           preferred_element_type=jnp.float32)
                   + qh_b + b1_ref[...])
        h2 = _gelu(jnp.dot(h1, w2_ref[...],
                           preferred_element_type=jnp.float32) + b2_ref[...])
        logits = jnp.dot(h2, w3_ref[...],
                         preferred_element_type=jnp.float32) + b3_ref[...]
        probs_ref[0, pl.ds(base, CHUNK), :] = _softmax_rows(logits)[:, :C]
        return carry

    jax.lax.fori_loop(0, pl.cdiv(n, CHUNK), _chunk, 0)

    col = jax.lax.broadcasted_iota(jnp.int32, (1, S), 1)
    tinst_ref[pl.ds(b, 1)] = jnp.where(col < n, tis_ref[pl.ds(b, 1)], 1.0)


def kernel(questions_embedding, context_segments_embedding,
           num_context_segments, target_agg_score, target_instance_score,
           W1, b1, W2, b2, W3, b3):
    b1_2d = b1.reshape(1, H1)
    b2_2d = b2.reshape(1, H2)
    w3p = jnp.pad(W3, ((0, 0), (0, CP - C)))
    b3p = jnp.concatenate([b3, jnp.full((CP - C,), -1e30, jnp.float32)])
    b3p = b3p.reshape(1, CP)

    grid_spec = pltpu.PrefetchScalarGridSpec(
        num_scalar_prefetch=1,
        grid=(G,),
        in_specs=[
            pl.BlockSpec((B, D), lambda b, n: (0, 0)),
            pl.BlockSpec((PB, S, D), lambda b, n: (b, 0, 0)),
            pl.BlockSpec((B, S), lambda b, n: (0, 0)),
            pl.BlockSpec((D, H1), lambda b, n: (0, 0)),
            pl.BlockSpec((D, H1), lambda b, n: (1, 0)),
            pl.BlockSpec((1, H1), lambda b, n: (0, 0)),
            pl.BlockSpec((H1, H2), lambda b, n: (0, 0)),
            pl.BlockSpec((1, H2), lambda b, n: (0, 0)),
            pl.BlockSpec((H2, CP), lambda b, n: (0, 0)),
            pl.BlockSpec((1, CP), lambda b, n: (0, 0)),
        ],
        out_specs=[
            pl.BlockSpec((PB, S, C), lambda b, n: (b, 0, 0)),
            pl.BlockSpec((B, S), lambda b, n: (0, 0)),
        ],
        scratch_shapes=[
            pltpu.VMEM((B, H1), jnp.float32),
            pltpu.VMEM((B, CP), jnp.float32),
        ],
    )

    probs, tinst = pl.pallas_call(
        _mlp_body,
        grid_spec=grid_spec,
        out_shape=[
            jax.ShapeDtypeStruct((B, S, C), jnp.float32),
            jax.ShapeDtypeStruct((B, S), jnp.float32),
        ],
    )(num_context_segments, questions_embedding, context_segments_embedding,
      target_instance_score, W1, W1, b1_2d, W2, b2_2d, w3p, b3p)

    return (target_agg_score, tinst, probs, num_context_segments)


# speedup vs baseline: 1.0708x; 1.0708x over previous
import jax
import jax.numpy as jnp
from jax.experimental import pallas as pl
from jax.experimental.pallas import tpu as pltpu

B, S, D = 8, 512, 768
C = 5
H1 = 768
H2 = 384
CP = 128

_C1 = 0.7978845608028654
_C2 = _C1 * 0.044715


def _gelu(x):
    t = jnp.tanh(x * (_C1 + _C2 * (x * x)))
    return x * (0.5 * t + 0.5)


def _mlp_body(nseg_ref, q_hbm, seg_hbm, tis_hbm, w1_hbm, b1_hbm, w2_hbm,
              b2_hbm, w3_hbm, b3_hbm, probs_hbm, tinst_hbm,
              q_v, seg_v, tis_v, w1_v, b1_v, w2_v, b2_v, w3_v, b3_v,
              probs_v, tinst_v, in_sem, seg_sem, out_sem):
    small = [
        pltpu.make_async_copy(q_hbm, q_v, in_sem.at[0]),
        pltpu.make_async_copy(w1_hbm, w1_v, in_sem.at[1]),
        pltpu.make_async_copy(b1_hbm, b1_v, in_sem.at[2]),
        pltpu.make_async_copy(w2_hbm, w2_v, in_sem.at[3]),
        pltpu.make_async_copy(b2_hbm, b2_v, in_sem.at[4]),
        pltpu.make_async_copy(w3_hbm, w3_v, in_sem.at[5]),
        pltpu.make_async_copy(b3_hbm, b3_v, in_sem.at[6]),
        pltpu.make_async_copy(tis_hbm, tis_v, in_sem.at[7]),
    ]
    for cp in small:
        cp.start()
    seg_cps = [
        pltpu.make_async_copy(seg_hbm.at[i], seg_v.at[i], seg_sem.at[i])
        for i in range(B)
    ]
    for cp in seg_cps:
        cp.start()
    for cp in small:
        cp.wait()

    qh = jnp.dot(q_v[...], w1_v[:D], preferred_element_type=jnp.float32)
    b1v = b1_v[...]
    b2v = b2_v[...]
    b3v = b3_v[...]

    row = jax.lax.broadcasted_iota(jnp.int32, (S, 1), 0)
    col = jax.lax.broadcasted_iota(jnp.int32, (1, S), 1)

    out_cps = []
    for b in range(B):
        n = nseg_ref[b]
        seg_cps[b].wait()
        x = jnp.where(row < n, seg_v[b], 0.0)
        h1 = _gelu(jnp.dot(x, w1_v[D:], preferred_element_type=jnp.float32)
                   + qh[b:b + 1] + b1v)
        h2 = _gelu(jnp.dot(h1, w2_v[...],
                           preferred_element_type=jnp.float32) + b2v)
        logits = jnp.dot(h2, w3_v[...],
                         preferred_element_type=jnp.float32) + b3v
        m = jnp.max(logits, axis=-1, keepdims=True)
        e = jnp.exp(logits - m)
        probs = e / jnp.sum(e, axis=-1, keepdims=True)
        probs_v[b] = probs[:, :C]
        cp = pltpu.make_async_copy(probs_v.at[b], probs_hbm.at[b],
                                   out_sem.at[b])
        cp.start()
        out_cps.append(cp)

        tinst_v[pl.ds(b, 1)] = jnp.where(col < n, tis_v[pl.ds(b, 1)], 1.0)

    ti_cp = pltpu.make_async_copy(tinst_v, tinst_hbm, out_sem.at[B])
    ti_cp.start()
    for cp in out_cps:
        cp.wait()
    ti_cp.wait()


def kernel(questions_embedding, context_segments_embedding,
           num_context_segments, target_agg_score, target_instance_score,
           W1, b1, W2, b2, W3, b3):
    b1_2d = b1.reshape(1, H1)
    b2_2d = b2.reshape(1, H2)
    w3p = jnp.pad(W3, ((0, 0), (0, CP - C)))
    b3p = jnp.concatenate([b3, jnp.full((CP - C,), -1e30, jnp.float32)])
    b3p = b3p.reshape(1, CP)

    any_spec = pl.BlockSpec(memory_space=pl.ANY)

    grid_spec = pltpu.PrefetchScalarGridSpec(
        num_scalar_prefetch=1,
        grid=(1,),
        in_specs=[any_spec] * 9,
        out_specs=[any_spec, any_spec],
        scratch_shapes=[
            pltpu.VMEM((B, D), jnp.float32),
            pltpu.VMEM((B, S, D), jnp.float32),
            pltpu.VMEM((B, S), jnp.float32),
            pltpu.VMEM((2 * D, H1), jnp.float32),
            pltpu.VMEM((1, H1), jnp.float32),
            pltpu.VMEM((H1, H2), jnp.float32),
            pltpu.VMEM((1, H2), jnp.float32),
            pltpu.VMEM((H2, CP), jnp.float32),
            pltpu.VMEM((1, CP), jnp.float32),
            pltpu.VMEM((B, S, C), jnp.float32),
            pltpu.VMEM((B, S), jnp.float32),
            pltpu.SemaphoreType.DMA((8,)),
            pltpu.SemaphoreType.DMA((B,)),
            pltpu.SemaphoreType.DMA((B + 1,)),
        ],
    )

    probs, tinst = pl.pallas_call(
        _mlp_body,
        grid_spec=grid_spec,
        out_shape=[
            jax.ShapeDtypeStruct((B, S, C), jnp.float32),
            jax.ShapeDtypeStruct((B, S), jnp.float32),
        ],
    )(num_context_segments, questions_embedding, context_segments_embedding,
      target_instance_score, W1, b1_2d, W2, b2_2d, w3p, b3p)

    return (target_agg_score, tinst, probs, num_context_segments)


# speedup vs baseline: 1.2450x; 1.1627x over previous
import jax
import jax.numpy as jnp
from jax.experimental import pallas as pl
from jax.experimental.pallas import tpu as pltpu

B, S, D = 8, 512, 768
C = 5
H1 = 768
H2 = 384
CP = 128

_C1 = 0.7978845608028654
_C2 = _C1 * 0.044715


def _gelu(x):
    t = jnp.tanh(x * (_C1 + _C2 * (x * x)))
    return x * (0.5 * t + 0.5)


def _mlp_body(nseg_ref, q_ref, seg_ref, tis_ref, w1q_ref, w1s_ref, b1_ref,
              w2_ref, b2_ref, w3_ref, b3_ref, probs_ref, tinst_ref, qh_s,
              w1s_bf, w2_bf, w3_bf):
    b = pl.program_id(0)

    @pl.when(b == 0)
    def _init():
        qh_s[...] = jnp.dot(q_ref[...], w1q_ref[...],
                            preferred_element_type=jnp.float32)
        w1s_bf[...] = w1s_ref[...].astype(jnp.bfloat16)
        w2_bf[...] = w2_ref[...].astype(jnp.bfloat16)
        w3_bf[...] = w3_ref[...].astype(jnp.bfloat16)

    n = nseg_ref[b]
    row = jax.lax.broadcasted_iota(jnp.int32, (S, 1), 0)
    x = jnp.where(row < n, seg_ref[0], 0.0).astype(jnp.bfloat16)

    h1 = _gelu(jnp.dot(x, w1s_bf[...], preferred_element_type=jnp.float32)
               + qh_s[pl.ds(b, 1)] + b1_ref[...])
    h2 = _gelu(jnp.dot(h1.astype(jnp.bfloat16), w2_bf[...],
                       preferred_element_type=jnp.float32) + b2_ref[...])
    logits = jnp.dot(h2.astype(jnp.bfloat16), w3_bf[...],
                     preferred_element_type=jnp.float32)
    logits = logits + b3_ref[...]
    m = jnp.max(logits, axis=-1, keepdims=True)
    e = jnp.exp(logits - m)
    probs = e / jnp.sum(e, axis=-1, keepdims=True)
    probs_ref[0] = probs[:, :C]

    col = jax.lax.broadcasted_iota(jnp.int32, (1, S), 1)
    tinst_ref[pl.ds(b, 1)] = jnp.where(col < n, tis_ref[pl.ds(b, 1)], 1.0)


def kernel(questions_embedding, context_segments_embedding,
           num_context_segments, target_agg_score, target_instance_score,
           W1, b1, W2, b2, W3, b3):
    b1_2d = b1.reshape(1, H1)
    b2_2d = b2.reshape(1, H2)
    w3p = jnp.pad(W3, ((0, 0), (0, CP - C)))
    b3p = jnp.concatenate([b3, jnp.full((CP - C,), -1e30, jnp.float32)])
    b3p = b3p.reshape(1, CP)

    grid_spec = pltpu.PrefetchScalarGridSpec(
        num_scalar_prefetch=1,
        grid=(B,),
        in_specs=[
            pl.BlockSpec((B, D), lambda b, n: (0, 0)),
            pl.BlockSpec((1, S, D), lambda b, n: (b, 0, 0)),
            pl.BlockSpec((B, S), lambda b, n: (0, 0)),
            pl.BlockSpec((D, H1), lambda b, n: (0, 0)),
            pl.BlockSpec((D, H1), lambda b, n: (1, 0)),
            pl.BlockSpec((1, H1), lambda b, n: (0, 0)),
            pl.BlockSpec((H1, H2), lambda b, n: (0, 0)),
            pl.BlockSpec((1, H2), lambda b, n: (0, 0)),
            pl.BlockSpec((H2, CP), lambda b, n: (0, 0)),
            pl.BlockSpec((1, CP), lambda b, n: (0, 0)),
        ],
        out_specs=[
            pl.BlockSpec((1, S, C), lambda b, n: (b, 0, 0)),
            pl.BlockSpec((B, S), lambda b, n: (0, 0)),
        ],
        scratch_shapes=[
            pltpu.VMEM((B, H1), jnp.float32),
            pltpu.VMEM((D, H1), jnp.bfloat16),
            pltpu.VMEM((H1, H2), jnp.bfloat16),
            pltpu.VMEM((H2, CP), jnp.bfloat16),
        ],
    )

    probs, tinst = pl.pallas_call(
        _mlp_body,
        grid_spec=grid_spec,
        out_shape=[
            jax.ShapeDtypeStruct((B, S, C), jnp.float32),
            jax.ShapeDtypeStruct((B, S), jnp.float32),
        ],
    )(num_context_segments, questions_embedding, context_segments_embedding,
      target_instance_score, W1, W1, b1_2d, W2, b2_2d, w3p, b3p)

    return (target_agg_score, tinst, probs, num_context_segments)
